# baseline (speedup 1.0000x reference)
import functools

import jax
import jax.numpy as jnp
from jax import lax
from jax.experimental import pallas as pl
from jax.experimental.pallas import tpu as pltpu
from jax.experimental.pallas import tpu_sc as plsc

B = 16384
K = 64
NC = 2
NS = 16
NW = NC * NS
BPW = B // NW
L = 16
GROUPS = BPW // L
CHUNK = 128
NCHUNK = BPW // CHUNK


def _rcfm_body(i_hbm, j_hbm, busr_hbm, bitm_hbm, usr_hbm, itm_hbm, c_hbm,
               out_hbm,
               idx_i, idx_j, rows_i, rows_j, bias_i, bias_j, c_v, out_v,
               sem_i, sem_j, sem_bi, sem_bj):
    wid = lax.axis_index("s") * NC + lax.axis_index("c")
    base = wid * BPW

    pltpu.sync_copy(i_hbm.at[pl.ds(base, BPW)], idx_i)
    pltpu.sync_copy(j_hbm.at[pl.ds(base, BPW)], idx_j)

    copies = []
    for q in range(NCHUNK):
        s = q * CHUNK
        copies.append(pltpu.async_copy(
            usr_hbm.at[idx_i.at[pl.ds(s, CHUNK)]],
            rows_i.at[pl.ds(s, CHUNK)], sem_i))
        copies.append(pltpu.async_copy(
            itm_hbm.at[idx_j.at[pl.ds(s, CHUNK)]],
            rows_j.at[pl.ds(s, CHUNK)], sem_j))
        copies.append(pltpu.async_copy(
            busr_hbm.at[idx_i.at[pl.ds(s, CHUNK)]],
            bias_i.at[pl.ds(s, CHUNK)], sem_bi))
        copies.append(pltpu.async_copy(
            bitm_hbm.at[idx_j.at[pl.ds(s, CHUNK)]],
            bias_j.at[pl.ds(s, CHUNK)], sem_bj))
    pltpu.sync_copy(c_hbm, c_v)
    for cp in copies:
        cp.wait()

    cvec = c_v[...]

    def group(g, _):
        ridx = g * L + lax.iota(jnp.int32, L)
        acc = bias_i[pl.ds(g * L, L)] + bias_j[pl.ds(g * L, L)] + cvec
        for k in range(K):
            colk = jnp.full((L,), k, jnp.int32)
            a = plsc.load_gather(rows_i, [ridx, colk])
            b = plsc.load_gather(rows_j, [ridx, colk])
            acc = acc + a * b
        out_v[pl.ds(g * L, L)] = acc
        return _

    lax.fori_loop(0, GROUPS, group, None)

    pltpu.sync_copy(out_v, out_hbm.at[pl.ds(base, BPW)])


@jax.jit
def kernel(i, j, y, busr, bitm, usr, itm, c):
    del y
    mesh = plsc.VectorSubcoreMesh(core_axis_name="c", subcore_axis_name="s")
    run = functools.partial(
        pl.kernel,
        mesh=mesh,
        out_type=jax.ShapeDtypeStruct((B,), jnp.float32),
        compiler_params=pltpu.CompilerParams(
            needs_layout_passes=False, use_tc_tiling_on_sc=False),
        scratch_types=[
            pltpu.VMEM((BPW,), jnp.int32),
            pltpu.VMEM((BPW,), jnp.int32),
            pltpu.VMEM((BPW, K), jnp.float32),
            pltpu.VMEM((BPW, K), jnp.float32),
            pltpu.VMEM((BPW,), jnp.float32),
            pltpu.VMEM((BPW,), jnp.float32),
            pltpu.VMEM((L,), jnp.float32),
            pltpu.VMEM((BPW,), jnp.float32),
            pltpu.SemaphoreType.DMA,
            pltpu.SemaphoreType.DMA,
            pltpu.SemaphoreType.DMA,
            pltpu.SemaphoreType.DMA,
        ],
    )(_rcfm_body)
    c16 = jnp.broadcast_to(c, (L,))
    return run(i.astype(jnp.int32), j.astype(jnp.int32),
               busr.reshape(-1), bitm.reshape(-1), usr, itm, c16)


# speedup vs baseline: 1.1606x; 1.1606x over previous
import functools

import jax
import jax.numpy as jnp
from jax import lax
from jax.experimental import pallas as pl
from jax.experimental.pallas import tpu as pltpu
from jax.experimental.pallas import tpu_sc as plsc

B = 16384
K = 64
NC = 2
NS = 16
NW = NC * NS
BPW = B // NW
L = 16
GROUPS = BPW // L
CHUNK = 128
NCHUNK = BPW // CHUNK


def _rcfm_body(i_hbm, j_hbm, busr_hbm, bitm_hbm, usr_hbm, itm_hbm, c_hbm,
               out_hbm,
               idx_i, idx_j, rows_i, rows_j, bias_i, bias_j, c_v, out_v,
               part, sem_iq, sem_jq, sem_bi, sem_bj):
    wid = lax.axis_index("s") * NC + lax.axis_index("c")
    base = wid * BPW

    pltpu.sync_copy(i_hbm.at[pl.ds(base, BPW)], idx_i)
    pltpu.sync_copy(j_hbm.at[pl.ds(base, BPW)], idx_j)

    row_cp = []
    for q in range(NCHUNK):
        s = q * CHUNK
        row_cp.append((
            pltpu.async_copy(usr_hbm.at[idx_i.at[pl.ds(s, CHUNK)]],
                             rows_i.at[pl.ds(s, CHUNK)], sem_iq[q]),
            pltpu.async_copy(itm_hbm.at[idx_j.at[pl.ds(s, CHUNK)]],
                             rows_j.at[pl.ds(s, CHUNK)], sem_jq[q]),
        ))
    bias_cp = []
    for q in range(NCHUNK):
        s = q * CHUNK
        bias_cp.append(pltpu.async_copy(
            busr_hbm.at[idx_i.at[pl.ds(s, CHUNK)]],
            bias_i.at[pl.ds(s, CHUNK)], sem_bi))
        bias_cp.append(pltpu.async_copy(
            bitm_hbm.at[idx_j.at[pl.ds(s, CHUNK)]],
            bias_j.at[pl.ds(s, CHUNK)], sem_bj))
    pltpu.sync_copy(c_hbm, c_v)
    for cp in bias_cp:
        cp.wait()

    cvec = c_v[...]
    iota = lax.iota(jnp.int32, L)
    gpc = CHUNK // L

    def group(g, _):
        for r in range(L):
            row = g * L + r
            tr = rows_i[row, pl.ds(0, L)] * rows_j[row, pl.ds(0, L)]
            for q in range(1, K // L):
                tr += (rows_i[row, pl.ds(q * L, L)]
                       * rows_j[row, pl.ds(q * L, L)])
            part[r] = tr
        acc = bias_i[pl.ds(g * L, L)] + bias_j[pl.ds(g * L, L)] + cvec
        for l in range(L):
            acc += plsc.load_gather(part, [iota, jnp.full((L,), l, jnp.int32)])
        out_v[pl.ds(g * L, L)] = acc
        return _

    for q in range(NCHUNK):
        for cp in row_cp[q]:
            cp.wait()
        lax.fori_loop(q * gpc, (q + 1) * gpc, group, None)

    pltpu.sync_copy(out_v, out_hbm.at[pl.ds(base, BPW)])


@jax.jit
def kernel(i, j, y, busr, bitm, usr, itm, c):
    del y
    mesh = plsc.VectorSubcoreMesh(core_axis_name="c", subcore_axis_name="s")
    run = functools.partial(
        pl.kernel,
        mesh=mesh,
        out_type=jax.ShapeDtypeStruct((B,), jnp.float32),
        compiler_params=pltpu.CompilerParams(
            needs_layout_passes=False, use_tc_tiling_on_sc=False),
        scratch_types=[
            pltpu.VMEM((BPW,), jnp.int32),
            pltpu.VMEM((BPW,), jnp.int32),
            pltpu.VMEM((BPW, K), jnp.float32),
            pltpu.VMEM((BPW, K), jnp.float32),
            pltpu.VMEM((BPW,), jnp.float32),
            pltpu.VMEM((BPW,), jnp.float32),
            pltpu.VMEM((L,), jnp.float32),
            pltpu.VMEM((BPW,), jnp.float32),
            pltpu.VMEM((L, L), jnp.float32),
            [pltpu.SemaphoreType.DMA] * NCHUNK,
            [pltpu.SemaphoreType.DMA] * NCHUNK,
            pltpu.SemaphoreType.DMA,
            pltpu.SemaphoreType.DMA,
        ],
    )(_rcfm_body)
    c16 = jnp.broadcast_to(c, (L,))
    return run(i.astype(jnp.int32), j.astype(jnp.int32),
               busr.reshape(-1), bitm.reshape(-1), usr, itm, c16)


# speedup vs baseline: 1.8584x; 1.6012x over previous
import functools

import jax
import jax.numpy as jnp
from jax import lax
from jax.experimental import pallas as pl
from jax.experimental.pallas import tpu as pltpu
from jax.experimental.pallas import tpu_sc as plsc

B = 16384
K = 64
N_VOCAB = 100000
NC = 2
NS = 16
NW = NC * NS
BPW = B // NW
L = 16
NPASS = K // NS
HALF = B // 2
CHUNK = 128
NCHUNK = BPW // CHUNK


def _gather_body(usrT_hbm, itmT_hbm, i_hbm, j_hbm, g_u, g_v,
                 frow, idxs, rowbuf):
    core = lax.axis_index("c")
    s = lax.axis_index("s")
    zeros = jnp.zeros((L,), jnp.int32)

    def side(tbl, ids, gout):
        pltpu.sync_copy(ids, idxs)
        for p in range(NPASS):
            f = p * NS + s
            pltpu.sync_copy(tbl.at[pl.ds(f, 1), :], frow)
            for h in range(2):
                def grp(g, _):
                    idxv = idxs[pl.ds(h * HALF + g * L, L)]
                    rowbuf[pl.ds(g * L, L)] = plsc.load_gather(
                        frow, [zeros, idxv])
                    return _
                lax.fori_loop(0, HALF // L, grp, None)
                pltpu.sync_copy(rowbuf, gout.at[pl.ds(f * B + h * HALF, HALF)])

    @pl.when(core == 0)
    def _():
        side(usrT_hbm, i_hbm, g_u)

    @pl.when(core == 1)
    def _():
        side(itmT_hbm, j_hbm, g_v)


def _dot_body(i_hbm, j_hbm, busr_hbm, bitm_hbm, g_u, g_v, c_hbm, out_hbm,
              idx_i, idx_j, ubuf, vbuf, bias_i, bias_j, c_v, out_v,
              sem_u, sem_v, sem_bi, sem_bj):
    core = lax.axis_index("c")
    s = lax.axis_index("s")
    wid = s * NC + core
    base = wid * BPW

    pltpu.sync_copy(i_hbm.at[pl.ds(base, BPW)], idx_i)
    pltpu.sync_copy(j_hbm.at[pl.ds(base, BPW)], idx_j)

    copies = []
    for f in range(K):
        copies.append(pltpu.async_copy(
            g_u.at[pl.ds(f * B + base, BPW)],
            ubuf.at[pl.ds(f * BPW, BPW)], sem_u))
        copies.append(pltpu.async_copy(
            g_v.at[pl.ds(f * B + base, BPW)],
            vbuf.at[pl.ds(f * BPW, BPW)], sem_v))
    for q in range(NCHUNK):
        sl = q * CHUNK
        copies.append(pltpu.async_copy(
            busr_hbm.at[idx_i.at[pl.ds(sl, CHUNK)]],
            bias_i.at[pl.ds(sl, CHUNK)], sem_bi))
        copies.append(pltpu.async_copy(
            bitm_hbm.at[idx_j.at[pl.ds(sl, CHUNK)]],
            bias_j.at[pl.ds(sl, CHUNK)], sem_bj))
    pltpu.sync_copy(c_hbm, c_v)
    for cp in copies:
        cp.wait()

    cvec = c_v[...]

    def grp(g, _):
        acc = bias_i[pl.ds(g * L, L)] + bias_j[pl.ds(g * L, L)] + cvec
        for f in range(K):
            acc += (ubuf[pl.ds(f * BPW + g * L, L)]
                    * vbuf[pl.ds(f * BPW + g * L, L)])
        out_v[pl.ds(g * L, L)] = acc
        return _

    lax.fori_loop(0, BPW // L, grp, None)
    pltpu.sync_copy(out_v, out_hbm.at[pl.ds(base, BPW)])


@jax.jit
def kernel(i, j, y, busr, bitm, usr, itm, c):
    del y
    mesh = plsc.VectorSubcoreMesh(core_axis_name="c", subcore_axis_name="s")
    gather_call = pl.kernel(
        _gather_body,
        mesh=mesh,
        out_type=(jax.ShapeDtypeStruct((K * B,), jnp.float32),
                  jax.ShapeDtypeStruct((K * B,), jnp.float32)),
        compiler_params=pltpu.CompilerParams(
            needs_layout_passes=False, use_tc_tiling_on_sc=True),
        scratch_types=[
            pltpu.VMEM((1, N_VOCAB), jnp.float32),
            pltpu.VMEM((B,), jnp.int32),
            pltpu.VMEM((HALF,), jnp.float32),
        ],
    )
    dot_call = pl.kernel(
        _dot_body,
        mesh=mesh,
        out_type=jax.ShapeDtypeStruct((B,), jnp.float32),
        compiler_params=pltpu.CompilerParams(
            needs_layout_passes=False, use_tc_tiling_on_sc=False),
        scratch_types=[
            pltpu.VMEM((BPW,), jnp.int32),
            pltpu.VMEM((BPW,), jnp.int32),
            pltpu.VMEM((K * BPW,), jnp.float32),
            pltpu.VMEM((K * BPW,), jnp.float32),
            pltpu.VMEM((BPW,), jnp.float32),
            pltpu.VMEM((BPW,), jnp.float32),
            pltpu.VMEM((L,), jnp.float32),
            pltpu.VMEM((BPW,), jnp.float32),
            pltpu.SemaphoreType.DMA,
            pltpu.SemaphoreType.DMA,
            pltpu.SemaphoreType.DMA,
            pltpu.SemaphoreType.DMA,
        ],
    )
    ii = i.astype(jnp.int32)
    jj = j.astype(jnp.int32)
    g_u, g_v = gather_call(usr.T, itm.T, ii, jj)
    c16 = jnp.broadcast_to(c, (L,))
    return dot_call(ii, jj, busr.reshape(-1), bitm.reshape(-1),
                    g_u, g_v, c16)


# speedup vs baseline: 1.8653x; 1.0037x over previous
import functools

import jax
import jax.numpy as jnp
from jax import lax
from jax.experimental import pallas as pl
from jax.experimental.pallas import tpu as pltpu
from jax.experimental.pallas import tpu_sc as plsc

B = 16384
K = 64
N_VOCAB = 100000
NC = 2
NS = 16
NW = NC * NS
BPW = B // NW
L = 16
NPASS = K // NS
QTR = B // 4
CHUNK = 128
NCHUNK = BPW // CHUNK


def _gather_body(usrT_hbm, itmT_hbm, i_hbm, j_hbm, g_u, g_v,
                 frow, idxs, rowbuf0, rowbuf1, sem_w):
    core = lax.axis_index("c")
    s = lax.axis_index("s")
    zeros = jnp.zeros((L,), jnp.int32)
    UNROLL = 4

    def side(tbl, ids, gout):
        pltpu.sync_copy(ids, idxs)
        rowbufs = (rowbuf0, rowbuf1)
        wr = []
        for p in range(NPASS):
            f = p * NS + s
            pltpu.sync_copy(tbl.at[pl.ds(f, 1), :], frow)
            for h in range(4):
                rowbuf = rowbufs[h % 2]
                if len(wr) >= 2:
                    wr.pop(0).wait()

                def grp(g4, _):
                    for u in range(UNROLL):
                        g = g4 * UNROLL + u
                        idxv = idxs[pl.ds(h * QTR + g * L, L)]
                        rowbuf[pl.ds(g * L, L)] = plsc.load_gather(
                            frow, [zeros, idxv])
                    return _
                lax.fori_loop(0, QTR // (L * UNROLL), grp, None)
                wr.append(pltpu.async_copy(
                    rowbuf, gout.at[pl.ds(f * B + h * QTR, QTR)], sem_w))
        for cp in wr:
            cp.wait()

    @pl.when(core == 0)
    def _():
        side(usrT_hbm, i_hbm, g_u)

    @pl.when(core == 1)
    def _():
        side(itmT_hbm, j_hbm, g_v)


def _dot_body(i_hbm, j_hbm, busr_hbm, bitm_hbm, g_u, g_v, c_hbm, out_hbm,
              idx_i, idx_j, ubuf, vbuf, bias_i, bias_j, c_v, out_v,
              sem_u, sem_v, sem_bi, sem_bj):
    core = lax.axis_index("c")
    s = lax.axis_index("s")
    wid = s * NC + core
    base = wid * BPW

    pltpu.sync_copy(i_hbm.at[pl.ds(base, BPW)], idx_i)
    pltpu.sync_copy(j_hbm.at[pl.ds(base, BPW)], idx_j)

    copies = []
    for f in range(K):
        copies.append(pltpu.async_copy(
            g_u.at[pl.ds(f * B + base, BPW)],
            ubuf.at[pl.ds(f * BPW, BPW)], sem_u))
        copies.append(pltpu.async_copy(
            g_v.at[pl.ds(f * B + base, BPW)],
            vbuf.at[pl.ds(f * BPW, BPW)], sem_v))
    for q in range(NCHUNK):
        sl = q * CHUNK
        copies.append(pltpu.async_copy(
            busr_hbm.at[idx_i.at[pl.ds(sl, CHUNK)]],
            bias_i.at[pl.ds(sl, CHUNK)], sem_bi))
        copies.append(pltpu.async_copy(
            bitm_hbm.at[idx_j.at[pl.ds(sl, CHUNK)]],
            bias_j.at[pl.ds(sl, CHUNK)], sem_bj))
    pltpu.sync_copy(c_hbm, c_v)
    for cp in copies:
        cp.wait()

    cvec = c_v[...]

    def grp(g, _):
        acc = bias_i[pl.ds(g * L, L)] + bias_j[pl.ds(g * L, L)] + cvec
        for f in range(K):
            acc += (ubuf[pl.ds(f * BPW + g * L, L)]
                    * vbuf[pl.ds(f * BPW + g * L, L)])
        out_v[pl.ds(g * L, L)] = acc
        return _

    lax.fori_loop(0, BPW // L, grp, None)
    pltpu.sync_copy(out_v, out_hbm.at[pl.ds(base, BPW)])


@jax.jit
def kernel(i, j, y, busr, bitm, usr, itm, c):
    del y
    mesh = plsc.VectorSubcoreMesh(core_axis_name="c", subcore_axis_name="s")
    gather_call = pl.kernel(
        _gather_body,
        mesh=mesh,
        out_type=(jax.ShapeDtypeStruct((K * B,), jnp.float32),
                  jax.ShapeDtypeStruct((K * B,), jnp.float32)),
        compiler_params=pltpu.CompilerParams(
            needs_layout_passes=False, use_tc_tiling_on_sc=True),
        scratch_types=[
            pltpu.VMEM((1, N_VOCAB), jnp.float32),
            pltpu.VMEM((B,), jnp.int32),
            pltpu.VMEM((QTR,), jnp.float32),
            pltpu.VMEM((QTR,), jnp.float32),
            pltpu.SemaphoreType.DMA,
        ],
    )
    dot_call = pl.kernel(
        _dot_body,
        mesh=mesh,
        out_type=jax.ShapeDtypeStruct((B,), jnp.float32),
        compiler_params=pltpu.CompilerParams(
            needs_layout_passes=False, use_tc_tiling_on_sc=False),
        scratch_types=[
            pltpu.VMEM((BPW,), jnp.int32),
            pltpu.VMEM((BPW,), jnp.int32),
            pltpu.VMEM((K * BPW,), jnp.float32),
            pltpu.VMEM((K * BPW,), jnp.float32),
            pltpu.VMEM((BPW,), jnp.float32),
            pltpu.VMEM((BPW,), jnp.float32),
            pltpu.VMEM((L,), jnp.float32),
            pltpu.VMEM((BPW,), jnp.float32),
            pltpu.SemaphoreType.DMA,
            pltpu.SemaphoreType.DMA,
            pltpu.SemaphoreType.DMA,
            pltpu.SemaphoreType.DMA,
        ],
    )
    ii = i.astype(jnp.int32)
    jj = j.astype(jnp.int32)
    g_u, g_v = gather_call(usr.T, itm.T, ii, jj)
    c16 = jnp.broadcast_to(c, (L,))
    return dot_call(ii, jj, busr.reshape(-1), bitm.reshape(-1),
                    g_u, g_v, c16)


# speedup vs baseline: 1.8876x; 1.0120x over previous
import functools

import jax
import jax.numpy as jnp
from jax import lax
from jax.experimental import pallas as pl
from jax.experimental.pallas import tpu as pltpu
from jax.experimental.pallas import tpu_sc as plsc

B = 16384
K = 64
N_VOCAB = 100000
NC = 2
NS = 16
NW = NC * NS
BPW = B // NW
L = 16
NPASS = K // NS
QTR = B // 4
CHUNK = 128
NCHUNK = BPW // CHUNK


def _gather_body(usrT_hbm, itmT_hbm, i_hbm, j_hbm, g_u, g_v,
                 frow, idxs, rowbuf0, rowbuf1, sem_w):
    core = lax.axis_index("c")
    s = lax.axis_index("s")
    zeros = jnp.zeros((L,), jnp.int32)
    UNROLL = 4

    def side(tbl, ids, gout):
        cp0 = pltpu.async_copy(tbl.at[pl.ds(s, 1), :], frow, sem_w)
        pltpu.sync_copy(ids, idxs)
        rowbufs = (rowbuf0, rowbuf1)
        wr = []
        for p in range(NPASS):
            f = p * NS + s
            if p == 0:
                cp0.wait()
            else:
                pltpu.sync_copy(tbl.at[pl.ds(f, 1), :], frow)
            for h in range(4):
                rowbuf = rowbufs[h % 2]
                if len(wr) >= 2:
                    wr.pop(0).wait()

                def grp(g4, _):
                    for u in range(UNROLL):
                        g = g4 * UNROLL + u
                        idxv = idxs[pl.ds(h * QTR + g * L, L)]
                        rowbuf[pl.ds(g * L, L)] = plsc.load_gather(
                            frow, [zeros, idxv])
                    return _
                lax.fori_loop(0, QTR // (L * UNROLL), grp, None)
                wr.append(pltpu.async_copy(
                    rowbuf, gout.at[pl.ds(f * B + h * QTR, QTR)], sem_w))
        for cp in wr:
            cp.wait()

    @pl.when(core == 0)
    def _():
        side(usrT_hbm, i_hbm, g_u)

    @pl.when(core == 1)
    def _():
        side(itmT_hbm, j_hbm, g_v)


def _dot_body(i_hbm, j_hbm, busr_hbm, bitm_hbm, g_u, g_v, c_hbm, out_hbm,
              idx_i, idx_j, ubuf, vbuf, bias_i, bias_j, c_v, out_v,
              sem_u, sem_v, sem_bi, sem_bj):
    core = lax.axis_index("c")
    s = lax.axis_index("s")
    wid = s * NC + core
    base = wid * BPW

    copies = []
    for f in range(K):
        copies.append(pltpu.async_copy(
            g_u.at[pl.ds(f * B + base, BPW)],
            ubuf.at[pl.ds(f * BPW, BPW)], sem_u))
        copies.append(pltpu.async_copy(
            g_v.at[pl.ds(f * B + base, BPW)],
            vbuf.at[pl.ds(f * BPW, BPW)], sem_v))
    pltpu.sync_copy(i_hbm.at[pl.ds(base, BPW)], idx_i)
    pltpu.sync_copy(j_hbm.at[pl.ds(base, BPW)], idx_j)
    for q in range(NCHUNK):
        sl = q * CHUNK
        copies.append(pltpu.async_copy(
            busr_hbm.at[idx_i.at[pl.ds(sl, CHUNK)]],
            bias_i.at[pl.ds(sl, CHUNK)], sem_bi))
        copies.append(pltpu.async_copy(
            bitm_hbm.at[idx_j.at[pl.ds(sl, CHUNK)]],
            bias_j.at[pl.ds(sl, CHUNK)], sem_bj))
    pltpu.sync_copy(c_hbm, c_v)
    for cp in copies:
        cp.wait()

    cvec = c_v[...]

    def grp(g, _):
        acc = bias_i[pl.ds(g * L, L)] + bias_j[pl.ds(g * L, L)] + cvec
        for f in range(K):
            acc += (ubuf[pl.ds(f * BPW + g * L, L)]
                    * vbuf[pl.ds(f * BPW + g * L, L)])
        out_v[pl.ds(g * L, L)] = acc
        return _

    lax.fori_loop(0, BPW // L, grp, None)
    pltpu.sync_copy(out_v, out_hbm.at[pl.ds(base, BPW)])


@jax.jit
def kernel(i, j, y, busr, bitm, usr, itm, c):
    del y
    mesh = plsc.VectorSubcoreMesh(core_axis_name="c", subcore_axis_name="s")
    gather_call = pl.kernel(
        _gather_body,
        mesh=mesh,
        out_type=(jax.ShapeDtypeStruct((K * B,), jnp.float32),
                  jax.ShapeDtypeStruct((K * B,), jnp.float32)),
        compiler_params=pltpu.CompilerParams(
            needs_layout_passes=False, use_tc_tiling_on_sc=True),
        scratch_types=[
            pltpu.VMEM((1, N_VOCAB), jnp.float32),
            pltpu.VMEM((B,), jnp.int32),
            pltpu.VMEM((QTR,), jnp.float32),
            pltpu.VMEM((QTR,), jnp.float32),
            pltpu.SemaphoreType.DMA,
        ],
    )
    dot_call = pl.kernel(
        _dot_body,
        mesh=mesh,
        out_type=jax.ShapeDtypeStruct((B,), jnp.float32),
        compiler_params=pltpu.CompilerParams(
            needs_layout_passes=False, use_tc_tiling_on_sc=False),
        scratch_types=[
            pltpu.VMEM((BPW,), jnp.int32),
            pltpu.VMEM((BPW,), jnp.int32),
            pltpu.VMEM((K * BPW,), jnp.float32),
            pltpu.VMEM((K * BPW,), jnp.float32),
            pltpu.VMEM((BPW,), jnp.float32),
            pltpu.VMEM((BPW,), jnp.float32),
            pltpu.VMEM((L,), jnp.float32),
            pltpu.VMEM((BPW,), jnp.float32),
            pltpu.SemaphoreType.DMA,
            pltpu.SemaphoreType.DMA,
            pltpu.SemaphoreType.DMA,
            pltpu.SemaphoreType.DMA,
        ],
    )
    ii = i.astype(jnp.int32)
    jj = j.astype(jnp.int32)
    g_u, g_v = gather_call(usr.T, itm.T, ii, jj)
    c16 = jnp.broadcast_to(c, (L,))
    return dot_call(ii, jj, busr.reshape(-1), bitm.reshape(-1),
                    g_u, g_v, c16)
